# baseline (speedup 1.0000x reference)
import functools

import jax
import jax.numpy as jnp
from jax import lax
from jax.experimental import pallas as pl
from jax.experimental.pallas import tpu as pltpu
from jax.experimental.pallas import tpu_sc as plsc

_N = 10000
_NCLS = 40
_D1 = 128
_D2 = 48
_NC = 2
_NS = 16
_NW = _NC * _NS
_NPAD = 10240
_RPT = _NPAD // _NS
_DEGW = 16
_C1, _NCH1 = 96, 108
_C2, _NCH2 = 128, 84


def _sc_agg(g, eds, zrows, d, c, nch, nb, gd, pd):
    ni = 2 * nb
    assert gd < nb and gd < pd and pd + nb - gd + 1 <= ni
    assert nch % ni == 0 and nch // ni >= 2
    mesh = plsc.VectorSubcoreMesh(core_axis_name="c", subcore_axis_name="s")

    @functools.partial(
        pl.kernel,
        out_type=jax.ShapeDtypeStruct((_NC, _NPAD, d), jnp.float32),
        mesh=mesh,
        scratch_types=[
            pltpu.VMEM((ni, 2, c), jnp.int32),
            pltpu.VMEM((nb, c, d), jnp.float32),
            pltpu.VMEM_SHARED((_NPAD, d), jnp.float32),
            pltpu.SemaphoreType.DMA,
            pltpu.SemaphoreType.DMA,
            pltpu.SemaphoreType.DMA,
        ],
        compiler_params=pltpu.CompilerParams(use_tc_tiling_on_sc=False),
    )
    def k(g_hbm, ed_hbm, z_hbm, out_hbm, ebuf, buf_v, acc_sh, isem, gsem, ssem):
        cid = lax.axis_index("c")
        sid = lax.axis_index("s")
        wid = sid * _NC + cid
        pltpu.sync_copy(z_hbm, buf_v.at[0])
        for r in range(_RPT // c):
            pltpu.sync_copy(buf_v.at[0], acc_sh.at[pl.ds(sid * _RPT + r * c, c)])
        if _RPT % c:
            pltpu.sync_copy(buf_v.at[0, pl.ds(0, _RPT % c)],
                            acc_sh.at[pl.ds(sid * _RPT + (_RPT // c) * c, _RPT % c)])
        plsc.subcore_barrier()

        def idx_issue(j, s):
            pltpu.async_copy(ed_hbm.at[wid, j], ebuf.at[s], isem)

        def idx_wait(j, s):
            pltpu.make_async_copy(ed_hbm.at[wid, j], ebuf.at[s], isem).wait()

        def gat_issue(s, bs):
            pltpu.async_copy(g_hbm.at[ebuf.at[s, 0]], buf_v.at[bs], gsem)

        def gat_wait(s, bs):
            pltpu.make_async_copy(g_hbm.at[ebuf.at[s, 0]], buf_v.at[bs], gsem).wait()

        def sca_issue(s, bs):
            pltpu.async_copy(buf_v.at[bs], acc_sh.at[ebuf.at[s, 1]], ssem, add=True)

        def sca_wait(s, bs):
            pltpu.make_async_copy(buf_v.at[bs], acc_sh.at[ebuf.at[s, 1]], ssem).wait()

        def emit(j, u, static):
            if not static or j + pd < nch:
                idx_issue(j + pd, (u + pd) % ni)
            if not static or j + gd < nch:
                if not static or j + gd - nb >= 0:
                    sca_wait((u + gd + nb) % ni, (u + gd) % nb)
                idx_wait(j + gd, (u + gd) % ni)
                gat_issue((u + gd) % ni, (u + gd) % nb)
            gat_wait(u % ni, u % nb)
            sca_issue(u % ni, u % nb)

        for t in range(pd):
            idx_issue(t, t)
        for t in range(gd):
            idx_wait(t, t)
            gat_issue(t, t)
        for u in range(ni):
            emit(u, u, True)

        def group(o, carry):
            j0 = (o + 1) * ni
            for u in range(ni):
                emit(j0 + u, u, False)
            return carry

        lax.fori_loop(0, nch // ni - 2, group, 0)
        for u in range(ni):
            emit(nch - ni + u, u, True)
        for t in range(nb):
            kk = nch - nb + t
            sca_wait(kk % ni, kk % nb)
        plsc.subcore_barrier()
        pltpu.sync_copy(acc_sh.at[pl.ds(sid * _RPT, _RPT)],
                        out_hbm.at[cid, pl.ds(sid * _RPT, _RPT)])

    return k(g, eds, zrows)


def _sc_degree(dsts, zo):
    mesh = plsc.VectorSubcoreMesh(core_axis_name="c", subcore_axis_name="s")

    @functools.partial(
        pl.kernel,
        out_type=jax.ShapeDtypeStruct((_NC, _NPAD, _DEGW), jnp.float32),
        mesh=mesh,
        scratch_types=[
            pltpu.VMEM((_NCH2, _C2), jnp.int32),
            pltpu.VMEM((_C2, _DEGW), jnp.float32),
            pltpu.VMEM_SHARED((_NPAD, _DEGW), jnp.float32),
        ],
        compiler_params=pltpu.CompilerParams(use_tc_tiling_on_sc=False),
    )
    def k(dst_hbm, zo_hbm, out_hbm, dst_v, val_v, acc_sh):
        cid = lax.axis_index("c")
        sid = lax.axis_index("s")
        wid = sid * _NC + cid
        pltpu.sync_copy(dst_hbm.at[wid], dst_v)
        pltpu.sync_copy(zo_hbm.at[0], val_v)
        for r in range(_RPT // _C2):
            pltpu.sync_copy(val_v, acc_sh.at[pl.ds(sid * _RPT + r * _C2, _C2)])
        plsc.subcore_barrier()
        pltpu.sync_copy(zo_hbm.at[1], val_v)

        def body(j, carry):
            pltpu.sync_copy(val_v, acc_sh.at[dst_v.at[j]], add=True)
            return carry

        lax.fori_loop(0, _NCH2, body, 0)
        plsc.subcore_barrier()
        pltpu.sync_copy(acc_sh.at[pl.ds(sid * _RPT, _RPT)],
                        out_hbm.at[cid, pl.ds(sid * _RPT, _RPT)])

    return k(dsts, zo)


def _dinv(degp_ref):
    deg = degp_ref[0] + degp_ref[1]
    return jnp.where(deg > 0, lax.rsqrt(jnp.maximum(deg, 1e-12)), 0.0)


def _tc1_kernel(degp_ref, x_ref, w1_ref, g_ref):
    dinv = _dinv(degp_ref)
    h = jnp.dot(x_ref[...], w1_ref[...], preferred_element_type=jnp.float32)
    g_ref[...] = dinv[:_N, 0:1] * h


def _tc2_kernel(p1_ref, degp_ref, b1_ref, w2_ref, g2_ref):
    dinv = _dinv(degp_ref)
    agg = p1_ref[0, :_N] + p1_ref[1, :_N]
    z = jnp.maximum(dinv[:_N, 0:1] * agg + b1_ref[...], 0.0)
    h2 = jnp.dot(z, w2_ref[...], preferred_element_type=jnp.float32)
    g2_ref[...] = dinv[:_N, 0:1] * h2


def _tc3_kernel(p2_ref, degp_ref, b2_ref, o_ref):
    dinv = _dinv(degp_ref)
    agg = p2_ref[0, :_N] + p2_ref[1, :_N]
    o = dinv[:_N, 0:1] * agg + b2_ref[...]
    col = lax.broadcasted_iota(jnp.int32, (_N, _D2), 1)
    logits = jnp.where(col < _NCLS, o, -jnp.inf)
    m = jnp.max(logits, axis=1, keepdims=True)
    ex = jnp.where(col < _NCLS, jnp.exp(o - m), 0.0)
    lse = jnp.log(jnp.sum(ex, axis=1, keepdims=True))
    o_ref[...] = o - m - lse


def _edge_layout(src, dst, c, nch):
    epad = _NW * nch * c
    pad_n = epad - src.shape[0]
    pad_dst = _N + jax.lax.rem(jnp.arange(pad_n, dtype=jnp.int32),
                               jnp.int32(_NPAD - _N))
    s = jnp.concatenate([src, jnp.zeros((pad_n,), jnp.int32)]).reshape(_NW, nch, c)
    t = jnp.concatenate([dst, pad_dst]).reshape(_NW, nch, c)
    return jnp.stack([s, t], axis=2), t


def kernel(x, edge_index, W1, b1, W2, b2):
    loops = jnp.arange(_N, dtype=jnp.int32)
    src = jnp.concatenate([edge_index[0].astype(jnp.int32), loops])
    dst = jnp.concatenate([edge_index[1].astype(jnp.int32), loops])
    eds1, _ = _edge_layout(src, dst, _C1, _NCH1)
    eds2, dsts2 = _edge_layout(src, dst, _C2, _NCH2)

    zo = jnp.zeros((2, _C2, _DEGW), jnp.float32).at[1, :, 0].set(1.0)
    z1 = jnp.zeros((_C1, _D1), jnp.float32)
    z2 = jnp.zeros((_C2, _D2), jnp.float32)
    w2p = jnp.pad(W2, ((0, 0), (0, _D2 - _NCLS)))
    b2p = jnp.pad(b2, (0, _D2 - _NCLS))

    degp = _sc_degree(dsts2, zo)

    g1 = pl.pallas_call(
        _tc1_kernel,
        out_shape=jax.ShapeDtypeStruct((_N, _D1), jnp.float32),
    )(degp, x, W1)

    p1 = _sc_agg(g1, eds1, z1, _D1, _C1, _NCH1, 3, 1, 2)

    g2 = pl.pallas_call(
        _tc2_kernel,
        out_shape=jax.ShapeDtypeStruct((_N, _D2), jnp.float32),
    )(p1, degp, b1, w2p)

    p2 = _sc_agg(g2, eds2, z2, _D2, _C2, _NCH2, 3, 1, 2)

    out = pl.pallas_call(
        _tc3_kernel,
        out_shape=jax.ShapeDtypeStruct((_N, _D2), jnp.float32),
    )(p2, degp, b2p)

    return out[:, :_NCLS]


# speedup vs baseline: 1.4295x; 1.4295x over previous
import functools

import jax
import jax.numpy as jnp
from jax import lax
from jax.experimental import pallas as pl
from jax.experimental.pallas import tpu as pltpu
from jax.experimental.pallas import tpu_sc as plsc

_N = 10000
_NCLS = 40
_D1 = 128
_D2 = 48
_NC = 2
_NS = 16
_NW = _NC * _NS
_NPAD = 10240
_RPT = _NPAD // _NS
_DEGW = 16
_C1, _NCH1 = 96, 108
_C2, _NCH2 = 96, 108


def _sc_agg(g, eds, zrows, d, c, nch, nb, gd, pd):
    ni = 2 * nb
    assert gd < nb and gd < pd and pd + nb - gd + 1 <= ni
    assert nch % ni == 0 and nch // ni >= 2
    mesh = plsc.VectorSubcoreMesh(core_axis_name="c", subcore_axis_name="s")

    @functools.partial(
        pl.kernel,
        out_type=jax.ShapeDtypeStruct((_NC, _NPAD, d), jnp.float32),
        mesh=mesh,
        scratch_types=[
            pltpu.VMEM((ni, 2, c), jnp.int32),
            pltpu.VMEM((nb, c, d), jnp.float32),
            pltpu.VMEM_SHARED((_NPAD, d), jnp.float32),
            pltpu.SemaphoreType.DMA,
            pltpu.SemaphoreType.DMA,
            pltpu.SemaphoreType.DMA,
        ],
        compiler_params=pltpu.CompilerParams(use_tc_tiling_on_sc=False),
    )
    def k(g_hbm, ed_hbm, z_hbm, out_hbm, ebuf, buf_v, acc_sh, isem, gsem, ssem):
        cid = lax.axis_index("c")
        sid = lax.axis_index("s")
        wid = sid * _NC + cid
        pltpu.sync_copy(z_hbm, buf_v.at[0])
        for r in range(_RPT // c):
            pltpu.sync_copy(buf_v.at[0], acc_sh.at[pl.ds(sid * _RPT + r * c, c)])
        if _RPT % c:
            pltpu.sync_copy(buf_v.at[0, pl.ds(0, _RPT % c)],
                            acc_sh.at[pl.ds(sid * _RPT + (_RPT // c) * c, _RPT % c)])
        plsc.subcore_barrier()

        def idx_issue(j, s):
            pltpu.async_copy(ed_hbm.at[wid, j], ebuf.at[s], isem)

        def idx_wait(j, s):
            pltpu.make_async_copy(ed_hbm.at[wid, j], ebuf.at[s], isem).wait()

        def gat_issue(s, bs):
            pltpu.async_copy(g_hbm.at[ebuf.at[s, 0]], buf_v.at[bs], gsem)

        def gat_wait(s, bs):
            pltpu.make_async_copy(g_hbm.at[ebuf.at[s, 0]], buf_v.at[bs], gsem).wait()

        def sca_issue(s, bs):
            pltpu.async_copy(buf_v.at[bs], acc_sh.at[ebuf.at[s, 1]], ssem, add=True)

        def sca_wait(s, bs):
            pltpu.make_async_copy(buf_v.at[bs], acc_sh.at[ebuf.at[s, 1]], ssem).wait()

        def emit(j, u, static):
            if not static or j + pd < nch:
                idx_issue(j + pd, (u + pd) % ni)
            if not static or j + gd < nch:
                if not static or j + gd - nb >= 0:
                    sca_wait((u + gd + nb) % ni, (u + gd) % nb)
                idx_wait(j + gd, (u + gd) % ni)
                gat_issue((u + gd) % ni, (u + gd) % nb)
            gat_wait(u % ni, u % nb)
            sca_issue(u % ni, u % nb)

        for t in range(pd):
            idx_issue(t, t)
        for t in range(gd):
            idx_wait(t, t)
            gat_issue(t, t)
        for u in range(ni):
            emit(u, u, True)

        def group(o, carry):
            j0 = (o + 1) * ni
            for u in range(ni):
                emit(j0 + u, u, False)
            return carry

        lax.fori_loop(0, nch // ni - 2, group, 0)
        for u in range(ni):
            emit(nch - ni + u, u, True)
        for t in range(nb):
            kk = nch - nb + t
            sca_wait(kk % ni, kk % nb)
        plsc.subcore_barrier()
        pltpu.sync_copy(acc_sh.at[pl.ds(sid * _RPT, _RPT)],
                        out_hbm.at[cid, pl.ds(sid * _RPT, _RPT)])

    return k(g, eds, zrows)


def _sc_degree(dsts, zo):
    mesh = plsc.VectorSubcoreMesh(core_axis_name="c", subcore_axis_name="s")

    @functools.partial(
        pl.kernel,
        out_type=jax.ShapeDtypeStruct((_NC, _NPAD, _DEGW), jnp.float32),
        mesh=mesh,
        scratch_types=[
            pltpu.VMEM((_NCH2, _C2), jnp.int32),
            pltpu.VMEM((_C2, _DEGW), jnp.float32),
            pltpu.VMEM_SHARED((_NPAD, _DEGW), jnp.float32),
        ],
        compiler_params=pltpu.CompilerParams(use_tc_tiling_on_sc=False),
    )
    def k(dst_hbm, zo_hbm, out_hbm, dst_v, val_v, acc_sh):
        cid = lax.axis_index("c")
        sid = lax.axis_index("s")
        wid = sid * _NC + cid
        pltpu.sync_copy(dst_hbm.at[wid], dst_v)
        pltpu.sync_copy(zo_hbm.at[0], val_v)
        for r in range(_RPT // _C2):
            pltpu.sync_copy(val_v, acc_sh.at[pl.ds(sid * _RPT + r * _C2, _C2)])
        if _RPT % _C2:
            pltpu.sync_copy(val_v.at[pl.ds(0, _RPT % _C2)],
                            acc_sh.at[pl.ds(sid * _RPT + (_RPT // _C2) * _C2,
                                            _RPT % _C2)])
        plsc.subcore_barrier()
        pltpu.sync_copy(zo_hbm.at[1], val_v)

        def body(j, carry):
            pltpu.sync_copy(val_v, acc_sh.at[dst_v.at[j]], add=True)
            return carry

        lax.fori_loop(0, _NCH2, body, 0)
        plsc.subcore_barrier()
        pltpu.sync_copy(acc_sh.at[pl.ds(sid * _RPT, _RPT)],
                        out_hbm.at[cid, pl.ds(sid * _RPT, _RPT)])

    return k(dsts, zo)


def _dinv(degp_ref):
    deg = degp_ref[0] + degp_ref[1]
    return jnp.where(deg > 0, lax.rsqrt(jnp.maximum(deg, 1e-12)), 0.0)


def _tc1_kernel(degp_ref, x_ref, w1_ref, g_ref):
    dinv = _dinv(degp_ref)
    h = jnp.dot(x_ref[...], w1_ref[...], preferred_element_type=jnp.float32)
    g_ref[...] = dinv[:_N, 0:1] * h


def _tc2_kernel(p1_ref, degp_ref, b1_ref, w2_ref, g2_ref):
    dinv = _dinv(degp_ref)
    agg = p1_ref[0, :_N] + p1_ref[1, :_N]
    z = jnp.maximum(dinv[:_N, 0:1] * agg + b1_ref[...], 0.0)
    h2 = jnp.dot(z, w2_ref[...], preferred_element_type=jnp.float32)
    g2_ref[...] = dinv[:_N, 0:1] * h2


def _tc3_kernel(p2_ref, degp_ref, b2_ref, o_ref):
    dinv = _dinv(degp_ref)
    agg = p2_ref[0, :_N] + p2_ref[1, :_N]
    o = dinv[:_N, 0:1] * agg + b2_ref[...]
    col = lax.broadcasted_iota(jnp.int32, (_N, _D2), 1)
    logits = jnp.where(col < _NCLS, o, -jnp.inf)
    m = jnp.max(logits, axis=1, keepdims=True)
    ex = jnp.where(col < _NCLS, jnp.exp(o - m), 0.0)
    lse = jnp.log(jnp.sum(ex, axis=1, keepdims=True))
    o_ref[...] = o - m - lse


def _edge_layout(src, dst, c, nch):
    epad = _NW * nch * c
    pad_n = epad - src.shape[0]
    pad_dst = _N + jax.lax.rem(jnp.arange(pad_n, dtype=jnp.int32),
                               jnp.int32(_NPAD - _N))
    s = jnp.concatenate([src, jnp.zeros((pad_n,), jnp.int32)]).reshape(_NW, nch, c)
    t = jnp.concatenate([dst, pad_dst]).reshape(_NW, nch, c)
    return jnp.stack([s, t], axis=2), t


def kernel(x, edge_index, W1, b1, W2, b2):
    loops = jnp.arange(_N, dtype=jnp.int32)
    src = jnp.concatenate([edge_index[0].astype(jnp.int32), loops])
    dst = jnp.concatenate([edge_index[1].astype(jnp.int32), loops])
    eds1, _ = _edge_layout(src, dst, _C1, _NCH1)
    eds2, dsts2 = _edge_layout(src, dst, _C2, _NCH2)

    zo = jnp.zeros((2, _C2, _DEGW), jnp.float32).at[1, :, 0].set(1.0)
    z1 = jnp.zeros((_C1, _D1), jnp.float32)
    z2 = jnp.zeros((_C2, _D2), jnp.float32)
    w2p = jnp.pad(W2, ((0, 0), (0, _D2 - _NCLS)))
    b2p = jnp.pad(b2, (0, _D2 - _NCLS))

    degp = _sc_degree(dsts2, zo)

    g1 = pl.pallas_call(
        _tc1_kernel,
        out_shape=jax.ShapeDtypeStruct((_N, _D1), jnp.float32),
    )(degp, x, W1)

    p1 = _sc_agg(g1, eds1, z1, _D1, _C1, _NCH1, 3, 1, 2)

    g2 = pl.pallas_call(
        _tc2_kernel,
        out_shape=jax.ShapeDtypeStruct((_N, _D2), jnp.float32),
    )(p1, degp, b1, w2p)

    p2 = _sc_agg(g2, eds2, z2, _D2, _C2, _NCH2, 3, 1, 2)

    out = pl.pallas_call(
        _tc3_kernel,
        out_shape=jax.ShapeDtypeStruct((_N, _D2), jnp.float32),
    )(p2, degp, b2p)

    return out[:, :_NCLS]


# speedup vs baseline: 1.5819x; 1.1066x over previous
import functools

import jax
import jax.numpy as jnp
from jax import lax
from jax.experimental import pallas as pl
from jax.experimental.pallas import tpu as pltpu
from jax.experimental.pallas import tpu_sc as plsc

_N = 10000
_NCLS = 40
_D1 = 128
_D2 = 48
_NC = 2
_NS = 16
_NW = _NC * _NS
_NPAD = 10240
_RPT = _NPAD // _NS
_DEGW = 16
_C1, _NCH1 = 96, 108
_C2, _NCH2 = 96, 108


def _sc_agg(g, eds, zrows, d, c, nch, nb, gd, pd, dt=jnp.float32):
    ni = 2 * nb
    assert gd < nb and gd < pd and pd + nb - gd + 1 <= ni
    assert nch % ni == 0 and nch // ni >= 2
    mesh = plsc.VectorSubcoreMesh(core_axis_name="c", subcore_axis_name="s")

    @functools.partial(
        pl.kernel,
        out_type=jax.ShapeDtypeStruct((_NC, _NPAD, d), dt),
        mesh=mesh,
        scratch_types=[
            pltpu.VMEM((ni, 2, c), jnp.int32),
            pltpu.VMEM((nb, c, d), dt),
            pltpu.VMEM_SHARED((_NPAD, d), dt),
            pltpu.SemaphoreType.DMA,
            pltpu.SemaphoreType.DMA,
            pltpu.SemaphoreType.DMA,
        ],
        compiler_params=pltpu.CompilerParams(use_tc_tiling_on_sc=False),
    )
    def k(g_hbm, ed_hbm, z_hbm, out_hbm, ebuf, buf_v, acc_sh, isem, gsem, ssem):
        cid = lax.axis_index("c")
        sid = lax.axis_index("s")
        wid = sid * _NC + cid
        pltpu.sync_copy(z_hbm, buf_v.at[0])
        for r in range(_RPT // c):
            pltpu.sync_copy(buf_v.at[0], acc_sh.at[pl.ds(sid * _RPT + r * c, c)])
        if _RPT % c:
            pltpu.sync_copy(buf_v.at[0, pl.ds(0, _RPT % c)],
                            acc_sh.at[pl.ds(sid * _RPT + (_RPT // c) * c, _RPT % c)])
        plsc.subcore_barrier()

        def idx_issue(j, s):
            pltpu.async_copy(ed_hbm.at[wid, j], ebuf.at[s], isem)

        def idx_wait(j, s):
            pltpu.make_async_copy(ed_hbm.at[wid, j], ebuf.at[s], isem).wait()

        def gat_issue(s, bs):
            pltpu.async_copy(g_hbm.at[ebuf.at[s, 0]], buf_v.at[bs], gsem)

        def gat_wait(s, bs):
            pltpu.make_async_copy(g_hbm.at[ebuf.at[s, 0]], buf_v.at[bs], gsem).wait()

        def sca_issue(s, bs):
            pltpu.async_copy(buf_v.at[bs], acc_sh.at[ebuf.at[s, 1]], ssem, add=True)

        def sca_wait(s, bs):
            pltpu.make_async_copy(buf_v.at[bs], acc_sh.at[ebuf.at[s, 1]], ssem).wait()

        def emit(j, u, static):
            if not static or j + pd < nch:
                idx_issue(j + pd, (u + pd) % ni)
            if not static or j + gd < nch:
                if not static or j + gd - nb >= 0:
                    sca_wait((u + gd + nb) % ni, (u + gd) % nb)
                idx_wait(j + gd, (u + gd) % ni)
                gat_issue((u + gd) % ni, (u + gd) % nb)
            gat_wait(u % ni, u % nb)
            sca_issue(u % ni, u % nb)

        for t in range(pd):
            idx_issue(t, t)
        for t in range(gd):
            idx_wait(t, t)
            gat_issue(t, t)
        for u in range(ni):
            emit(u, u, True)

        def group(o, carry):
            j0 = (o + 1) * ni
            for u in range(ni):
                emit(j0 + u, u, False)
            return carry

        lax.fori_loop(0, nch // ni - 2, group, 0)
        for u in range(ni):
            emit(nch - ni + u, u, True)
        for t in range(nb):
            kk = nch - nb + t
            sca_wait(kk % ni, kk % nb)
        plsc.subcore_barrier()
        pltpu.sync_copy(acc_sh.at[pl.ds(sid * _RPT, _RPT)],
                        out_hbm.at[cid, pl.ds(sid * _RPT, _RPT)])

    return k(g, eds, zrows)


def _sc_degree(dsts, zo):
    mesh = plsc.VectorSubcoreMesh(core_axis_name="c", subcore_axis_name="s")

    @functools.partial(
        pl.kernel,
        out_type=jax.ShapeDtypeStruct((_NC, _NPAD, _DEGW), jnp.float32),
        mesh=mesh,
        scratch_types=[
            pltpu.VMEM((_NCH2, _C2), jnp.int32),
            pltpu.VMEM((_C2, _DEGW), jnp.float32),
            pltpu.VMEM_SHARED((_NPAD, _DEGW), jnp.float32),
        ],
        compiler_params=pltpu.CompilerParams(use_tc_tiling_on_sc=False),
    )
    def k(dst_hbm, zo_hbm, out_hbm, dst_v, val_v, acc_sh):
        cid = lax.axis_index("c")
        sid = lax.axis_index("s")
        wid = sid * _NC + cid
        pltpu.sync_copy(dst_hbm.at[wid], dst_v)
        pltpu.sync_copy(zo_hbm.at[0], val_v)
        for r in range(_RPT // _C2):
            pltpu.sync_copy(val_v, acc_sh.at[pl.ds(sid * _RPT + r * _C2, _C2)])
        if _RPT % _C2:
            pltpu.sync_copy(val_v.at[pl.ds(0, _RPT % _C2)],
                            acc_sh.at[pl.ds(sid * _RPT + (_RPT // _C2) * _C2,
                                            _RPT % _C2)])
        plsc.subcore_barrier()
        pltpu.sync_copy(zo_hbm.at[1], val_v)

        def body(j, carry):
            pltpu.sync_copy(val_v, acc_sh.at[dst_v.at[j]], add=True)
            return carry

        lax.fori_loop(0, _NCH2, body, 0)
        plsc.subcore_barrier()
        pltpu.sync_copy(acc_sh.at[pl.ds(sid * _RPT, _RPT)],
                        out_hbm.at[cid, pl.ds(sid * _RPT, _RPT)])

    return k(dsts, zo)


def _dinv(degp_ref):
    deg = degp_ref[0] + degp_ref[1]
    return jnp.where(deg > 0, lax.rsqrt(jnp.maximum(deg, 1e-12)), 0.0)


def _tc1_kernel(degp_ref, x_ref, w1_ref, g_ref):
    dinv = _dinv(degp_ref)
    h = jnp.dot(x_ref[...], w1_ref[...], preferred_element_type=jnp.float32)
    g_ref[...] = (dinv[:_N, 0:1] * h).astype(jnp.bfloat16)


def _tc2_kernel(p1_ref, degp_ref, b1_ref, w2_ref, g2_ref):
    dinv = _dinv(degp_ref)
    agg = (p1_ref[0, :_N].astype(jnp.float32)
           + p1_ref[1, :_N].astype(jnp.float32))
    z = jnp.maximum(dinv[:_N, 0:1] * agg + b1_ref[...], 0.0)
    h2 = jnp.dot(z, w2_ref[...], preferred_element_type=jnp.float32)
    g2_ref[...] = dinv[:_N, 0:1] * h2


def _tc3_kernel(p2_ref, degp_ref, b2_ref, o_ref):
    dinv = _dinv(degp_ref)
    agg = p2_ref[0, :_N] + p2_ref[1, :_N]
    o = dinv[:_N, 0:1] * agg + b2_ref[...]
    col = lax.broadcasted_iota(jnp.int32, (_N, _D2), 1)
    logits = jnp.where(col < _NCLS, o, -jnp.inf)
    m = jnp.max(logits, axis=1, keepdims=True)
    ex = jnp.where(col < _NCLS, jnp.exp(o - m), 0.0)
    lse = jnp.log(jnp.sum(ex, axis=1, keepdims=True))
    o_ref[...] = o - m - lse


def _edge_layout(src, dst, c, nch):
    epad = _NW * nch * c
    pad_n = epad - src.shape[0]
    pad_dst = _N + jax.lax.rem(jnp.arange(pad_n, dtype=jnp.int32),
                               jnp.int32(_NPAD - _N))
    s = jnp.concatenate([src, jnp.zeros((pad_n,), jnp.int32)]).reshape(_NW, nch, c)
    t = jnp.concatenate([dst, pad_dst]).reshape(_NW, nch, c)
    return jnp.stack([s, t], axis=2), t


def kernel(x, edge_index, W1, b1, W2, b2):
    loops = jnp.arange(_N, dtype=jnp.int32)
    src = jnp.concatenate([edge_index[0].astype(jnp.int32), loops])
    dst = jnp.concatenate([edge_index[1].astype(jnp.int32), loops])
    eds1, _ = _edge_layout(src, dst, _C1, _NCH1)
    eds2, dsts2 = _edge_layout(src, dst, _C2, _NCH2)

    zo = jnp.zeros((2, _C2, _DEGW), jnp.float32).at[1, :, 0].set(1.0)
    z1 = jnp.zeros((_C1, _D1), jnp.bfloat16)
    z2 = jnp.zeros((_C2, _D2), jnp.float32)
    w2p = jnp.pad(W2, ((0, 0), (0, _D2 - _NCLS)))
    b2p = jnp.pad(b2, (0, _D2 - _NCLS))

    degp = _sc_degree(dsts2, zo)

    g1 = pl.pallas_call(
        _tc1_kernel,
        out_shape=jax.ShapeDtypeStruct((_N, _D1), jnp.bfloat16),
    )(degp, x, W1)

    p1 = _sc_agg(g1, eds1, z1, _D1, _C1, _NCH1, 3, 1, 2, jnp.bfloat16)

    g2 = pl.pallas_call(
        _tc2_kernel,
        out_shape=jax.ShapeDtypeStruct((_N, _D2), jnp.float32),
    )(p1, degp, b1, w2p)

    p2 = _sc_agg(g2, eds2, z2, _D2, _C2, _NCH2, 3, 1, 2)

    out = pl.pallas_call(
        _tc3_kernel,
        out_shape=jax.ShapeDtypeStruct((_N, _D2), jnp.float32),
    )(p2, degp, b2p)

    return out[:, :_NCLS]


# speedup vs baseline: 1.7439x; 1.1024x over previous
import functools

import jax
import jax.numpy as jnp
from jax import lax
from jax.experimental import pallas as pl
from jax.experimental.pallas import tpu as pltpu
from jax.experimental.pallas import tpu_sc as plsc

_N = 10000
_NCLS = 40
_D1 = 128
_D2 = 48
_NC = 2
_NS = 16
_NW = _NC * _NS
_NPAD = 10240
_RPT = _NPAD // _NS
_DEGW = 16
_C1, _NCH1 = 96, 108
_C2, _NCH2 = 96, 108


def _sc_agg(g, eds, zrows, d, c, nch, nb, gd, pd, dt=jnp.float32):
    ni = 2 * nb
    assert gd < nb and gd < pd and pd + nb - gd + 1 <= ni
    assert nch % ni == 0 and nch // ni >= 2
    mesh = plsc.VectorSubcoreMesh(core_axis_name="c", subcore_axis_name="s")

    @functools.partial(
        pl.kernel,
        out_type=jax.ShapeDtypeStruct((_NC, _NPAD, d), dt),
        mesh=mesh,
        scratch_types=[
            pltpu.VMEM((ni, 2, c), jnp.int32),
            pltpu.VMEM((nb, c, d), dt),
            pltpu.VMEM_SHARED((_NPAD, d), dt),
            pltpu.SemaphoreType.DMA,
            pltpu.SemaphoreType.DMA,
            pltpu.SemaphoreType.DMA,
        ],
        compiler_params=pltpu.CompilerParams(use_tc_tiling_on_sc=False),
    )
    def k(g_hbm, ed_hbm, z_hbm, out_hbm, ebuf, buf_v, acc_sh, isem, gsem, ssem):
        cid = lax.axis_index("c")
        sid = lax.axis_index("s")
        wid = sid * _NC + cid
        pltpu.sync_copy(z_hbm, buf_v.at[0])
        for r in range(_RPT // c):
            pltpu.sync_copy(buf_v.at[0], acc_sh.at[pl.ds(sid * _RPT + r * c, c)])
        if _RPT % c:
            pltpu.sync_copy(buf_v.at[0, pl.ds(0, _RPT % c)],
                            acc_sh.at[pl.ds(sid * _RPT + (_RPT // c) * c, _RPT % c)])
        plsc.subcore_barrier()

        def idx_issue(j, s):
            pltpu.async_copy(ed_hbm.at[wid, j], ebuf.at[s], isem)

        def idx_wait(j, s):
            pltpu.make_async_copy(ed_hbm.at[wid, j], ebuf.at[s], isem).wait()

        def gat_issue(s, bs):
            pltpu.async_copy(g_hbm.at[ebuf.at[s, 0]], buf_v.at[bs], gsem)

        def gat_wait(s, bs):
            pltpu.make_async_copy(g_hbm.at[ebuf.at[s, 0]], buf_v.at[bs], gsem).wait()

        def sca_issue(s, bs):
            pltpu.async_copy(buf_v.at[bs], acc_sh.at[ebuf.at[s, 1]], ssem, add=True)

        def sca_wait(s, bs):
            pltpu.make_async_copy(buf_v.at[bs], acc_sh.at[ebuf.at[s, 1]], ssem).wait()

        def emit(j, u, static):
            if not static or j + pd < nch:
                idx_issue(j + pd, (u + pd) % ni)
            if not static or j + gd < nch:
                if not static or j + gd - nb >= 0:
                    sca_wait((u + gd + nb) % ni, (u + gd) % nb)
                idx_wait(j + gd, (u + gd) % ni)
                gat_issue((u + gd) % ni, (u + gd) % nb)
            gat_wait(u % ni, u % nb)
            sca_issue(u % ni, u % nb)

        for t in range(pd):
            idx_issue(t, t)
        for t in range(gd):
            idx_wait(t, t)
            gat_issue(t, t)
        for u in range(ni):
            emit(u, u, True)

        def group(o, carry):
            j0 = (o + 1) * ni
            for u in range(ni):
                emit(j0 + u, u, False)
            return carry

        lax.fori_loop(0, nch // ni - 2, group, 0)
        for u in range(ni):
            emit(nch - ni + u, u, True)
        for t in range(nb):
            kk = nch - nb + t
            sca_wait(kk % ni, kk % nb)
        plsc.subcore_barrier()
        pltpu.sync_copy(acc_sh.at[pl.ds(sid * _RPT, _RPT)],
                        out_hbm.at[cid, pl.ds(sid * _RPT, _RPT)])

    return k(g, eds, zrows)


def _sc_degree(dsts, zo):
    mesh = plsc.VectorSubcoreMesh(core_axis_name="c", subcore_axis_name="s")

    @functools.partial(
        pl.kernel,
        out_type=jax.ShapeDtypeStruct((_NC, _NPAD, _DEGW), jnp.float32),
        mesh=mesh,
        scratch_types=[
            pltpu.VMEM((_NCH2, _C2), jnp.int32),
            pltpu.VMEM((_C2, _DEGW), jnp.float32),
            pltpu.VMEM_SHARED((_NPAD, _DEGW), jnp.float32),
        ],
        compiler_params=pltpu.CompilerParams(use_tc_tiling_on_sc=False),
    )
    def k(dst_hbm, zo_hbm, out_hbm, dst_v, val_v, acc_sh):
        cid = lax.axis_index("c")
        sid = lax.axis_index("s")
        wid = sid * _NC + cid
        pltpu.sync_copy(dst_hbm.at[wid], dst_v)
        pltpu.sync_copy(zo_hbm.at[0], val_v)
        for r in range(_RPT // _C2):
            pltpu.sync_copy(val_v, acc_sh.at[pl.ds(sid * _RPT + r * _C2, _C2)])
        if _RPT % _C2:
            pltpu.sync_copy(val_v.at[pl.ds(0, _RPT % _C2)],
                            acc_sh.at[pl.ds(sid * _RPT + (_RPT // _C2) * _C2,
                                            _RPT % _C2)])
        plsc.subcore_barrier()
        pltpu.sync_copy(zo_hbm.at[1], val_v)

        def body(j, carry):
            pltpu.sync_copy(val_v, acc_sh.at[dst_v.at[j]], add=True)
            return carry

        lax.fori_loop(0, _NCH2, body, 0)
        plsc.subcore_barrier()
        pltpu.sync_copy(acc_sh.at[pl.ds(sid * _RPT, _RPT)],
                        out_hbm.at[cid, pl.ds(sid * _RPT, _RPT)])

    return k(dsts, zo)


def _dinv(degp_ref):
    deg = degp_ref[0] + degp_ref[1]
    return jnp.where(deg > 0, lax.rsqrt(jnp.maximum(deg, 1e-12)), 0.0)


def _tc1_kernel(degp_ref, x_ref, w1_ref, g_ref):
    dinv = _dinv(degp_ref)
    h = jnp.dot(x_ref[...], w1_ref[...], preferred_element_type=jnp.float32)
    g_ref[...] = (dinv[:_N, 0:1] * h).astype(jnp.bfloat16)


def _tc2_kernel(p1_ref, degp_ref, b1_ref, w2_ref, g2_ref):
    dinv = _dinv(degp_ref)
    agg = (p1_ref[0, :_N].astype(jnp.float32)
           + p1_ref[1, :_N].astype(jnp.float32))
    z = jnp.maximum(dinv[:_N, 0:1] * agg + b1_ref[...], 0.0)
    h2 = jnp.dot(z, w2_ref[...], preferred_element_type=jnp.float32)
    g2_ref[...] = (dinv[:_N, 0:1] * h2).astype(jnp.bfloat16)


def _tc3_kernel(p2_ref, degp_ref, b2_ref, o_ref):
    dinv = _dinv(degp_ref)
    agg = (p2_ref[0, :_N].astype(jnp.float32)
           + p2_ref[1, :_N].astype(jnp.float32))
    o = dinv[:_N, 0:1] * agg + b2_ref[...]
    col = lax.broadcasted_iota(jnp.int32, (_N, _D2), 1)
    logits = jnp.where(col < _NCLS, o, -jnp.inf)
    m = jnp.max(logits, axis=1, keepdims=True)
    ex = jnp.where(col < _NCLS, jnp.exp(o - m), 0.0)
    lse = jnp.log(jnp.sum(ex, axis=1, keepdims=True))
    o_ref[...] = o - m - lse


def _edge_layout(src, dst, c, nch):
    epad = _NW * nch * c
    pad_n = epad - src.shape[0]
    pad_dst = _N + jax.lax.rem(jnp.arange(pad_n, dtype=jnp.int32),
                               jnp.int32(_NPAD - _N))
    s = jnp.concatenate([src, jnp.zeros((pad_n,), jnp.int32)]).reshape(_NW, nch, c)
    t = jnp.concatenate([dst, pad_dst]).reshape(_NW, nch, c)
    return jnp.stack([s, t], axis=2), t


def kernel(x, edge_index, W1, b1, W2, b2):
    loops = jnp.arange(_N, dtype=jnp.int32)
    src = jnp.concatenate([edge_index[0].astype(jnp.int32), loops])
    dst = jnp.concatenate([edge_index[1].astype(jnp.int32), loops])
    eds1, _ = _edge_layout(src, dst, _C1, _NCH1)
    eds2, dsts2 = _edge_layout(src, dst, _C2, _NCH2)

    zo = jnp.zeros((2, _C2, _DEGW), jnp.float32).at[1, :, 0].set(1.0)
    z1 = jnp.zeros((_C1, _D1), jnp.bfloat16)
    z2 = jnp.zeros((_C2, _D2), jnp.bfloat16)
    w2p = jnp.pad(W2, ((0, 0), (0, _D2 - _NCLS)))
    b2p = jnp.pad(b2, (0, _D2 - _NCLS))

    degp = _sc_degree(dsts2, zo)

    g1 = pl.pallas_call(
        _tc1_kernel,
        out_shape=jax.ShapeDtypeStruct((_N, _D1), jnp.bfloat16),
    )(degp, x, W1)

    p1 = _sc_agg(g1, eds1, z1, _D1, _C1, _NCH1, 3, 1, 2, jnp.bfloat16)

    g2 = pl.pallas_call(
        _tc2_kernel,
        out_shape=jax.ShapeDtypeStruct((_N, _D2), jnp.bfloat16),
    )(p1, degp, b1, w2p)

    p2 = _sc_agg(g2, eds2, z2, _D2, _C2, _NCH2, 3, 1, 2, jnp.bfloat16)

    out = pl.pallas_call(
        _tc3_kernel,
        out_shape=jax.ShapeDtypeStruct((_N, _D2), jnp.float32),
    )(p2, degp, b2p)

    return out[:, :_NCLS]
